# baseline (speedup 1.0000x reference)
import jax
import jax.numpy as jnp
from jax.experimental import pallas as pl
from jax.experimental.pallas import tpu as pltpu

N = 4096
B = 8
K = 5
G = 10
C = 10
RB = 512


def _l1_first_body(L_ref, x_ref, y_ref, Lbf_ref):
    Lb = L_ref[...].astype(jnp.bfloat16)
    Lbf_ref[...] = Lb
    y_ref[...] = jnp.dot(Lb, x_ref[...].astype(jnp.bfloat16),
                         preferred_element_type=jnp.float32)


def _apply_rec_body(L_ref, t_ref, tprev_ref, out_ref):
    out_ref[...] = 2.0 * jnp.dot(L_ref[...],
                                 t_ref[...].astype(jnp.bfloat16),
                                 preferred_element_type=jnp.float32) \
        - tprev_ref[...]


def _mix_first_body(L_ref, t0, t1, t2, t3, t4, w_ref, b_ref, y_ref, h_ref):
    acc = jnp.dot(t0[...], w_ref[0], preferred_element_type=jnp.float32)
    acc += jnp.dot(t1[...], w_ref[1], preferred_element_type=jnp.float32)
    acc += jnp.dot(t2[...], w_ref[2], preferred_element_type=jnp.float32)
    acc += jnp.dot(t3[...], w_ref[3], preferred_element_type=jnp.float32)
    acc += jnp.dot(t4[...], w_ref[4], preferred_element_type=jnp.float32)
    h = jax.nn.relu(acc + b_ref[...])
    h_ref[...] = h
    y_ref[...] = jnp.dot(L_ref[...], h.astype(jnp.bfloat16),
                         preferred_element_type=jnp.float32)


def _mix_body(t0, t1, t2, t3, t4, w_ref, b_ref, out_ref):
    acc = jnp.dot(t0[...], w_ref[0], preferred_element_type=jnp.float32)
    acc += jnp.dot(t1[...], w_ref[1], preferred_element_type=jnp.float32)
    acc += jnp.dot(t2[...], w_ref[2], preferred_element_type=jnp.float32)
    acc += jnp.dot(t3[...], w_ref[3], preferred_element_type=jnp.float32)
    acc += jnp.dot(t4[...], w_ref[4], preferred_element_type=jnp.float32)
    out_ref[...] = jax.nn.relu(acc + b_ref[...])


def _fc_body(h_ref, fcw_ref, fcb_ref, out_ref):
    logits = jnp.dot(h_ref[...], fcw_ref[...],
                     preferred_element_type=jnp.float32) + fcb_ref[...]
    m = jnp.max(logits, axis=1, keepdims=True)
    s = jnp.log(jnp.sum(jnp.exp(logits - m), axis=1, keepdims=True))
    out_ref[...] = logits - (m + s)


def _grid_specs(cols, n_full, extra_blocked=None):
    specs = [pl.BlockSpec((RB, N), lambda i: (i, 0))]
    specs += [pl.BlockSpec((N, cols), lambda i: (0, 0))] * n_full
    if extra_blocked is not None:
        specs.append(pl.BlockSpec((RB, extra_blocked), lambda i: (i, 0)))
    return specs


def _lapply_rec(Lbf, t, tprev):
    cols = t.shape[1]
    return pl.pallas_call(
        _apply_rec_body,
        grid=(N // RB,),
        in_specs=_grid_specs(cols, 1, extra_blocked=cols),
        out_specs=pl.BlockSpec((RB, cols), lambda i: (i, 0)),
        out_shape=jax.ShapeDtypeStruct((N, cols), jnp.float32),
    )(Lbf, t, tprev)


@jax.jit
def kernel(x, L, W1, b1, W2, b2, W3, b3, fc_w, fc_b):
    x0 = x[:, :, 0].T
    eyeB = jnp.eye(B, dtype=jnp.float32)
    w1_bd = jnp.einsum('ab,kfg->kafbg', eyeB, W1).reshape(K, B, B * G)
    w2_bd = jnp.einsum('ab,kfg->kafbg', eyeB, W2).reshape(K, B * G, B * G)
    w3_bd = jnp.einsum('ab,kfg->kafbg', eyeB, W3).reshape(K, B * G, B * G)
    bb1 = jnp.tile(b1, B)[None, :]
    bb2 = jnp.tile(b2, B)[None, :]
    bb3 = jnp.tile(b3, B)[None, :]

    t1, Lbf = pl.pallas_call(
        _l1_first_body,
        grid=(N // RB,),
        in_specs=_grid_specs(B, 1),
        out_specs=[pl.BlockSpec((RB, B), lambda i: (i, 0)),
                   pl.BlockSpec((RB, N), lambda i: (i, 0))],
        out_shape=[jax.ShapeDtypeStruct((N, B), jnp.float32),
                   jax.ShapeDtypeStruct((N, N), jnp.bfloat16)],
    )(L, x0)
    ts = [x0, t1]
    for _ in range(2, K):
        ts.append(_lapply_rec(Lbf, ts[-1], ts[-2]))

    for w_bd, bias in ((w1_bd, bb1), (w2_bd, bb2)):
        y1, h = pl.pallas_call(
            _mix_first_body,
            grid=(N // RB,),
            in_specs=_grid_specs(ts[0].shape[1], 5) + [
                pl.BlockSpec(w_bd.shape, lambda i: (0, 0, 0)),
                pl.BlockSpec((1, B * G), lambda i: (0, 0)),
            ],
            out_specs=[pl.BlockSpec((RB, B * G), lambda i: (i, 0)),
                       pl.BlockSpec((N, B * G), lambda i: (0, 0))],
            out_shape=[jax.ShapeDtypeStruct((N, B * G), jnp.float32),
                       jax.ShapeDtypeStruct((N, B * G), jnp.float32)],
        )(Lbf, *ts, w_bd, bias)
        ts = [h, y1]
        for _ in range(2, K):
            ts.append(_lapply_rec(Lbf, ts[-1], ts[-2]))

    h3 = pl.pallas_call(
        _mix_body,
        in_specs=[pl.BlockSpec((N, B * G), lambda: (0, 0))] * 5 + [
            pl.BlockSpec(w3_bd.shape, lambda: (0, 0, 0)),
            pl.BlockSpec((1, B * G), lambda: (0, 0)),
        ],
        out_specs=pl.BlockSpec((N, B * G), lambda: (0, 0)),
        out_shape=jax.ShapeDtypeStruct((N, B * G), jnp.float32),
    )(*ts, w3_bd, bb3)

    ht = h3.reshape(N, B, G).transpose(1, 0, 2).reshape(B, N * G)
    return pl.pallas_call(
        _fc_body,
        out_shape=jax.ShapeDtypeStruct((B, C), jnp.float32),
    )(ht, fc_w, fc_b[None, :])


# speedup vs baseline: 1.5556x; 1.5556x over previous
import jax
import jax.numpy as jnp
from jax.experimental import pallas as pl
from jax.experimental.pallas import tpu as pltpu

N = 4096
B = 8
K = 5
G = 10
C = 10
W = B * G
CB = 1024
NSTEP = 12
NBLK = N // CB


def _cast_body(L_ref, out_ref):
    out_ref[...] = L_ref[...].astype(jnp.bfloat16)


def _gcn_body(L_ref, x0_ref, wseq_ref, bseq_ref, wlast_ref, blast_ref,
              h_ref, P0, P1, Tbf, acc):
    s = pl.program_id(0)
    j = pl.program_id(1)
    p = jax.lax.rem(s, 4)

    @pl.when(jnp.logical_and(s == 0, j == 0))
    def _():
        t0 = x0_ref[...]
        P0[...] = t0
        Tbf[...] = t0.astype(jnp.bfloat16)
        acc[...] = jnp.dot(wseq_ref[0, 1], t0,
                           preferred_element_type=jnp.float32)

    @pl.when(jnp.logical_and(jnp.logical_and(p == 0, s > 0), j == 0))
    def _():
        a = acc[...] + jnp.dot(wseq_ref[0, 0], P0[...],
                               preferred_element_type=jnp.float32)
        t0 = jax.nn.relu(a + bseq_ref[0])
        P0[...] = t0
        Tbf[...] = t0.astype(jnp.bfloat16)
        acc[...] = jnp.dot(wseq_ref[0, 1], t0,
                           preferred_element_type=jnp.float32)

    @pl.when(jnp.logical_and(jax.lax.rem(p, 2) == 1, j == 0))
    def _():
        acc[...] += jnp.dot(wseq_ref[0, 0], P1[...],
                            preferred_element_type=jnp.float32)
        Tbf[...] = P1[...].astype(jnp.bfloat16)

    @pl.when(jnp.logical_and(p == 2, j == 0))
    def _():
        acc[...] += jnp.dot(wseq_ref[0, 0], P0[...],
                            preferred_element_type=jnp.float32)
        Tbf[...] = P0[...].astype(jnp.bfloat16)

    D = jnp.dot(Tbf[...], L_ref[...], preferred_element_type=jnp.float32)
    blk = pl.ds(j * CB, CB)

    @pl.when(jax.lax.rem(p, 2) == 0)
    def _():
        P1[:, blk] = jnp.where(p == 0, D, 2.0 * D - P1[:, blk])

    @pl.when(jax.lax.rem(p, 2) == 1)
    def _():
        P0[:, blk] = 2.0 * D - P0[:, blk]

    @pl.when(s == NSTEP - 1)
    def _():
        t4b = P0[:, blk]
        h_ref[:, blk] = jax.nn.relu(
            acc[:, blk] + jnp.dot(wlast_ref[...], t4b,
                                  preferred_element_type=jnp.float32)
            + blast_ref[...])


def _fc_body(h_ref, fcw_ref, fcb_ref, out_ref):
    logits = jnp.dot(h_ref[...], fcw_ref[...],
                     preferred_element_type=jnp.float32) + fcb_ref[...]
    m = jnp.max(logits, axis=1, keepdims=True)
    s = jnp.log(jnp.sum(jnp.exp(logits - m), axis=1, keepdims=True))
    out_ref[...] = logits - (m + s)


@jax.jit
def kernel(x, L, W1, b1, W2, b2, W3, b3, fc_w, fc_b):
    f32 = jnp.float32
    eyeB = jnp.eye(B, dtype=f32)
    w1_bd = jnp.einsum('ab,kfg->kafbg', eyeB, W1).reshape(K, B, W)
    w2_bd = jnp.einsum('ab,kfg->kafbg', eyeB, W2).reshape(K, W, W)
    w3_bd = jnp.einsum('ab,kfg->kafbg', eyeB, W3).reshape(K, W, W)
    w1t = jnp.pad(jnp.transpose(w1_bd, (0, 2, 1)), ((0, 0), (0, 0), (0, W - B)))
    w2t = jnp.transpose(w2_bd, (0, 2, 1))
    w3t = jnp.transpose(w3_bd, (0, 2, 1))
    wt = [w1t, w2t, w3t]
    biases = [jnp.tile(b1, B)[:, None], jnp.tile(b2, B)[:, None],
              jnp.tile(b3, B)[:, None]]

    zw = jnp.zeros((W, W), f32)
    wseq, bseq = [], []
    for s in range(NSTEP):
        p, l = s % 4, s // 4
        if p == 0:
            wseq.append(jnp.stack([zw if s == 0 else wt[l - 1][4], wt[l][0]]))
            bseq.append(jnp.zeros((W, 1), f32) if s == 0 else biases[l - 1])
        else:
            wseq.append(jnp.stack([wt[l][p], zw]))
            bseq.append(jnp.zeros((W, 1), f32))
    wseq = jnp.stack(wseq)
    bseq = jnp.stack(bseq)

    x0 = jnp.pad(x[:, :, 0], ((0, W - B), (0, 0)))

    Lbf = pl.pallas_call(
        _cast_body,
        grid=(NBLK,),
        in_specs=[pl.BlockSpec((N, CB), lambda j: (0, j))],
        out_specs=pl.BlockSpec((N, CB), lambda j: (0, j)),
        out_shape=jax.ShapeDtypeStruct((N, N), jnp.bfloat16),
    )(L)

    h3t = pl.pallas_call(
        _gcn_body,
        grid=(NSTEP, NBLK),
        in_specs=[
            pl.BlockSpec((N, CB), lambda s, j: (0, j)),
            pl.BlockSpec((W, N), lambda s, j: (0, 0)),
            pl.BlockSpec((1, 2, W, W), lambda s, j: (s, 0, 0, 0)),
            pl.BlockSpec((1, W, 1), lambda s, j: (s, 0, 0)),
            pl.BlockSpec((W, W), lambda s, j: (0, 0)),
            pl.BlockSpec((W, 1), lambda s, j: (0, 0)),
        ],
        out_specs=pl.BlockSpec((W, N), lambda s, j: (0, 0)),
        out_shape=jax.ShapeDtypeStruct((W, N), f32),
        scratch_shapes=[
            pltpu.VMEM((W, N), f32),
            pltpu.VMEM((W, N), f32),
            pltpu.VMEM((W, N), jnp.bfloat16),
            pltpu.VMEM((W, N), f32),
        ],
    )(Lbf, x0, wseq, bseq, wt[2][4], biases[2])

    ht = h3t.reshape(B, G, N).transpose(0, 2, 1).reshape(B, N * G)
    return pl.pallas_call(
        _fc_body,
        out_shape=jax.ShapeDtypeStruct((B, C), jnp.float32),
    )(ht, fc_w, fc_b[None, :])


# speedup vs baseline: 1.6022x; 1.0300x over previous
import jax
import jax.numpy as jnp
from jax.experimental import pallas as pl
from jax.experimental.pallas import tpu as pltpu

N = 4096
B = 8
K = 5
G = 10
C = 10
W = B * G
CB = 2048
NSTEP = 12
NBLK = N // CB


def _cast_body(L_ref, out_ref):
    out_ref[...] = L_ref[...].astype(jnp.bfloat16)


def _gcn_body(L_ref, x0_ref, wseq_ref, bseq_ref, wlast_ref, blast_ref,
              h_ref, P0, P1, Tbf, acc):
    s = pl.program_id(0)
    j = pl.program_id(1)
    p = jax.lax.rem(s, 4)

    @pl.when(jnp.logical_and(s == 0, j == 0))
    def _():
        t0 = x0_ref[...]
        P0[...] = t0
        Tbf[...] = t0.astype(jnp.bfloat16)
        acc[...] = jnp.dot(wseq_ref[0, 1], t0,
                           preferred_element_type=jnp.float32)

    @pl.when(jnp.logical_and(jnp.logical_and(p == 0, s > 0), j == 0))
    def _():
        a = acc[...] + jnp.dot(wseq_ref[0, 0], P0[...],
                               preferred_element_type=jnp.float32)
        t0 = jax.nn.relu(a + bseq_ref[0])
        P0[...] = t0
        Tbf[...] = t0.astype(jnp.bfloat16)
        acc[...] = jnp.dot(wseq_ref[0, 1], t0,
                           preferred_element_type=jnp.float32)

    @pl.when(jnp.logical_and(jax.lax.rem(p, 2) == 1, j == 0))
    def _():
        acc[...] += jnp.dot(wseq_ref[0, 0], P1[...],
                            preferred_element_type=jnp.float32)
        Tbf[...] = P1[...].astype(jnp.bfloat16)

    @pl.when(jnp.logical_and(p == 2, j == 0))
    def _():
        acc[...] += jnp.dot(wseq_ref[0, 0], P0[...],
                            preferred_element_type=jnp.float32)
        Tbf[...] = P0[...].astype(jnp.bfloat16)

    D = jnp.dot(Tbf[...], L_ref[...], preferred_element_type=jnp.float32)
    blk = pl.ds(j * CB, CB)

    @pl.when(jax.lax.rem(p, 2) == 0)
    def _():
        P1[:, blk] = jnp.where(p == 0, D, 2.0 * D - P1[:, blk])

    @pl.when(jax.lax.rem(p, 2) == 1)
    def _():
        P0[:, blk] = 2.0 * D - P0[:, blk]

    @pl.when(s == NSTEP - 1)
    def _():
        t4b = P0[:, blk]
        h_ref[:, blk] = jax.nn.relu(
            acc[:, blk] + jnp.dot(wlast_ref[...], t4b,
                                  preferred_element_type=jnp.float32)
            + blast_ref[...])


def _fc_body(h_ref, fcw_ref, fcb_ref, out_ref):
    logits = jnp.dot(h_ref[...], fcw_ref[...],
                     preferred_element_type=jnp.float32) + fcb_ref[...]
    m = jnp.max(logits, axis=1, keepdims=True)
    s = jnp.log(jnp.sum(jnp.exp(logits - m), axis=1, keepdims=True))
    out_ref[...] = logits - (m + s)


@jax.jit
def kernel(x, L, W1, b1, W2, b2, W3, b3, fc_w, fc_b):
    f32 = jnp.float32
    eyeB = jnp.eye(B, dtype=f32)
    w1_bd = jnp.einsum('ab,kfg->kafbg', eyeB, W1).reshape(K, B, W)
    w2_bd = jnp.einsum('ab,kfg->kafbg', eyeB, W2).reshape(K, W, W)
    w3_bd = jnp.einsum('ab,kfg->kafbg', eyeB, W3).reshape(K, W, W)
    w1t = jnp.pad(jnp.transpose(w1_bd, (0, 2, 1)), ((0, 0), (0, 0), (0, W - B)))
    w2t = jnp.transpose(w2_bd, (0, 2, 1))
    w3t = jnp.transpose(w3_bd, (0, 2, 1))
    wt = [w1t, w2t, w3t]
    biases = [jnp.tile(b1, B)[:, None], jnp.tile(b2, B)[:, None],
              jnp.tile(b3, B)[:, None]]

    zw = jnp.zeros((W, W), f32)
    wseq, bseq = [], []
    for s in range(NSTEP):
        p, l = s % 4, s // 4
        if p == 0:
            wseq.append(jnp.stack([zw if s == 0 else wt[l - 1][4], wt[l][0]]))
            bseq.append(jnp.zeros((W, 1), f32) if s == 0 else biases[l - 1])
        else:
            wseq.append(jnp.stack([wt[l][p], zw]))
            bseq.append(jnp.zeros((W, 1), f32))
    wseq = jnp.stack(wseq)
    bseq = jnp.stack(bseq)

    x0 = jnp.pad(x[:, :, 0], ((0, W - B), (0, 0)))

    CCB = 512
    Lbf = pl.pallas_call(
        _cast_body,
        grid=(N // CCB,),
        in_specs=[pl.BlockSpec((N, CCB), lambda j: (0, j))],
        out_specs=pl.BlockSpec((N, CCB), lambda j: (0, j)),
        out_shape=jax.ShapeDtypeStruct((N, N), jnp.bfloat16),
    )(L)

    h3t = pl.pallas_call(
        _gcn_body,
        grid=(NSTEP, NBLK),
        in_specs=[
            pl.BlockSpec((N, CB), lambda s, j: (0, j)),
            pl.BlockSpec((W, N), lambda s, j: (0, 0)),
            pl.BlockSpec((1, 2, W, W), lambda s, j: (s, 0, 0, 0)),
            pl.BlockSpec((1, W, 1), lambda s, j: (s, 0, 0)),
            pl.BlockSpec((W, W), lambda s, j: (0, 0)),
            pl.BlockSpec((W, 1), lambda s, j: (0, 0)),
        ],
        out_specs=pl.BlockSpec((W, N), lambda s, j: (0, 0)),
        out_shape=jax.ShapeDtypeStruct((W, N), f32),
        scratch_shapes=[
            pltpu.VMEM((W, N), f32),
            pltpu.VMEM((W, N), f32),
            pltpu.VMEM((W, N), jnp.bfloat16),
            pltpu.VMEM((W, N), f32),
        ],
    )(Lbf, x0, wseq, bseq, wt[2][4], biases[2])

    ht = h3t.reshape(B, G, N).transpose(0, 2, 1).reshape(B, N * G)
    return pl.pallas_call(
        _fc_body,
        out_shape=jax.ShapeDtypeStruct((B, C), jnp.float32),
    )(ht, fc_w, fc_b[None, :])


# speedup vs baseline: 1.7144x; 1.0701x over previous
import jax
import jax.numpy as jnp
from jax.experimental import pallas as pl
from jax.experimental.pallas import tpu as pltpu

N = 4096
B = 8
K = 5
G = 10
C = 10
W = B * G
CB = 2048
CCB = 512
NSTEP = 11
NBLK = N // CB


def _cast_t1_body(L_ref, x0_ref, Lbf_ref, y1_ref):
    Lb = L_ref[...].astype(jnp.bfloat16)
    Lbf_ref[...] = Lb
    y1_ref[...] = jnp.dot(x0_ref[...].astype(jnp.bfloat16), Lb,
                          preferred_element_type=jnp.float32)


def _gcn_body(L_ref, x0_ref, y1_ref, wseq_ref, bseq_ref, wlast_ref,
              blast_ref, h_ref, P0, P1, Tbf, acc):
    s = pl.program_id(0)
    j = pl.program_id(1)
    q = jax.lax.rem(s + 1, 4)

    @pl.when(jnp.logical_and(s == 0, j == 0))
    def _():
        t0 = x0_ref[...]
        t1 = y1_ref[...]
        P0[...] = t0
        P1[...] = t1
        Tbf[...] = t1.astype(jnp.bfloat16)
        acc[...] = jnp.dot(wseq_ref[0, 1], t0,
                           preferred_element_type=jnp.float32) \
            + jnp.dot(wseq_ref[0, 0], t1,
                      preferred_element_type=jnp.float32)

    @pl.when(jnp.logical_and(q == 0, j == 0))
    def _():
        a = acc[...] + jnp.dot(wseq_ref[0, 0], P0[...],
                               preferred_element_type=jnp.float32)
        t0 = jax.nn.relu(a + bseq_ref[0])
        P0[...] = t0
        Tbf[...] = t0.astype(jnp.bfloat16)
        acc[...] = jnp.dot(wseq_ref[0, 1], t0,
                           preferred_element_type=jnp.float32)

    @pl.when(jnp.logical_and(jnp.logical_and(jax.lax.rem(q, 2) == 1, s > 0),
                             j == 0))
    def _():
        acc[...] += jnp.dot(wseq_ref[0, 0], P1[...],
                            preferred_element_type=jnp.float32)
        Tbf[...] = P1[...].astype(jnp.bfloat16)

    @pl.when(jnp.logical_and(q == 2, j == 0))
    def _():
        acc[...] += jnp.dot(wseq_ref[0, 0], P0[...],
                            preferred_element_type=jnp.float32)
        Tbf[...] = P0[...].astype(jnp.bfloat16)

    D = jnp.dot(Tbf[...], L_ref[...], preferred_element_type=jnp.float32)
    blk = pl.ds(j * CB, CB)

    @pl.when(jax.lax.rem(q, 2) == 0)
    def _():
        P1[:, blk] = jnp.where(q == 0, D, 2.0 * D - P1[:, blk])

    @pl.when(jax.lax.rem(q, 2) == 1)
    def _():
        P0[:, blk] = 2.0 * D - P0[:, blk]

    @pl.when(s == NSTEP - 1)
    def _():
        t4b = P0[:, blk]
        h_ref[:, blk] = jax.nn.relu(
            acc[:, blk] + jnp.dot(wlast_ref[...], t4b,
                                  preferred_element_type=jnp.float32)
            + blast_ref[...])


def _fc_body(h_ref, fcw_ref, fcb_ref, out_ref):
    logits = jnp.dot(h_ref[...], fcw_ref[...],
                     preferred_element_type=jnp.float32) + fcb_ref[...]
    m = jnp.max(logits, axis=1, keepdims=True)
    s = jnp.log(jnp.sum(jnp.exp(logits - m), axis=1, keepdims=True))
    out_ref[...] = logits - (m + s)


@jax.jit
def kernel(x, L, W1, b1, W2, b2, W3, b3, fc_w, fc_b):
    f32 = jnp.float32
    eyeB = jnp.eye(B, dtype=f32)
    w1_bd = jnp.einsum('ab,kfg->kafbg', eyeB, W1).reshape(K, B, W)
    w2_bd = jnp.einsum('ab,kfg->kafbg', eyeB, W2).reshape(K, W, W)
    w3_bd = jnp.einsum('ab,kfg->kafbg', eyeB, W3).reshape(K, W, W)
    w1t = jnp.pad(jnp.transpose(w1_bd, (0, 2, 1)), ((0, 0), (0, 0), (0, W - B)))
    w2t = jnp.transpose(w2_bd, (0, 2, 1))
    w3t = jnp.transpose(w3_bd, (0, 2, 1))
    wt = [w1t, w2t, w3t]
    biases = [jnp.tile(b1, B)[:, None], jnp.tile(b2, B)[:, None],
              jnp.tile(b3, B)[:, None]]

    zw = jnp.zeros((W, W), f32)
    wseq, bseq = [], []
    for s in range(NSTEP):
        q, l = (s + 1) % 4, (s + 1) // 4
        if s == 0:
            wseq.append(jnp.stack([wt[0][1], wt[0][0]]))
            bseq.append(jnp.zeros((W, 1), f32))
        elif q == 0:
            wseq.append(jnp.stack([wt[l - 1][4], wt[l][0]]))
            bseq.append(biases[l - 1])
        else:
            wseq.append(jnp.stack([wt[l][q], zw]))
            bseq.append(jnp.zeros((W, 1), f32))
    wseq = jnp.stack(wseq)
    bseq = jnp.stack(bseq)

    x0 = jnp.pad(x[:, :, 0], ((0, W - B), (0, 0)))

    Lbf, y1 = pl.pallas_call(
        _cast_t1_body,
        grid=(N // CCB,),
        in_specs=[pl.BlockSpec((N, CCB), lambda j: (0, j)),
                  pl.BlockSpec((W, N), lambda j: (0, 0))],
        out_specs=[pl.BlockSpec((N, CCB), lambda j: (0, j)),
                   pl.BlockSpec((W, CCB), lambda j: (0, j))],
        out_shape=[jax.ShapeDtypeStruct((N, N), jnp.bfloat16),
                   jax.ShapeDtypeStruct((W, N), f32)],
    )(L, x0)

    h3t = pl.pallas_call(
        _gcn_body,
        grid=(NSTEP, NBLK),
        in_specs=[
            pl.BlockSpec((N, CB), lambda s, j: (0, j)),
            pl.BlockSpec((W, N), lambda s, j: (0, 0)),
            pl.BlockSpec((W, N), lambda s, j: (0, 0)),
            pl.BlockSpec((1, 2, W, W), lambda s, j: (s, 0, 0, 0)),
            pl.BlockSpec((1, W, 1), lambda s, j: (s, 0, 0)),
            pl.BlockSpec((W, W), lambda s, j: (0, 0)),
            pl.BlockSpec((W, 1), lambda s, j: (0, 0)),
        ],
        out_specs=pl.BlockSpec((W, N), lambda s, j: (0, 0)),
        out_shape=jax.ShapeDtypeStruct((W, N), f32),
        scratch_shapes=[
            pltpu.VMEM((W, N), f32),
            pltpu.VMEM((W, N), f32),
            pltpu.VMEM((W, N), jnp.bfloat16),
            pltpu.VMEM((W, N), f32),
        ],
    )(Lbf, x0, y1, wseq, bseq, wt[2][4], biases[2])

    ht = h3t.reshape(B, G, N).transpose(0, 2, 1).reshape(B, N * G)
    return pl.pallas_call(
        _fc_body,
        out_shape=jax.ShapeDtypeStruct((B, C), jnp.float32),
    )(ht, fc_w, fc_b[None, :])
